# baseline (device time: 55954 ns/iter reference)
import functools

import jax
import jax.numpy as jnp
from jax import lax
from jax.experimental import pallas as pl
from jax.experimental.pallas import tpu as pltpu

N_DEV = 16
B, SQ, SKV = 2, 128, 128
HQ_LOC, DH = 4, 64
D_MODEL = 512
HD_LOC = HQ_LOC * DH
N_STEPS = 4


def kernel(x, Wq, K_ext, V_ext, Wo):
    def body(x_ref, wq_ref, k_ref, v_ref, wo_ref, out_ref,
             acc_ref, recv_ref, send_sems, recv_sems):
        my = lax.axis_index("i")

        barrier_sem = pltpu.get_barrier_semaphore()
        for s in range(N_STEPS):
            pl.semaphore_signal(
                barrier_sem, inc=1,
                device_id=(my ^ (1 << s),),
                device_id_type=pl.DeviceIdType.MESH,
            )
        pl.semaphore_wait(barrier_sem, N_STEPS)

        col0 = my * HD_LOC
        wq_cols = wq_ref[:, pl.ds(col0, HD_LOC)]
        wo_rows = wo_ref[pl.ds(col0, HD_LOC), :]

        row_blk = lax.broadcasted_iota(jnp.int32, (SQ, SKV), 0) // 64
        col_blk = lax.broadcasted_iota(jnp.int32, (SQ, SKV), 1) // 64
        mask = col_blk <= row_blk

        for b in range(B):
            q_b = jnp.dot(x_ref[b], wq_cols,
                          preferred_element_type=jnp.float32)
            ctx_heads = []
            for h in range(HQ_LOC):
                q = q_b[:, h * DH:(h + 1) * DH]
                k = k_ref[b, :, h, :]
                v = v_ref[b, :, h, :]
                scores = lax.dot_general(
                    q, k, (((1,), (1,)), ((), ())),
                    preferred_element_type=jnp.float32) * 0.125
                scores = jnp.where(mask, scores, -1e9)
                m = jnp.max(scores, axis=1, keepdims=True)
                w = jnp.exp(scores - m)
                w = w / jnp.sum(w, axis=1, keepdims=True)
                ctx_heads.append(
                    jnp.dot(w, v, preferred_element_type=jnp.float32))
            ctx_b = jnp.concatenate(ctx_heads, axis=1)
            acc_ref[b] = jnp.dot(ctx_b, wo_rows,
                                 preferred_element_type=jnp.float32)

        for s in range(N_STEPS):
            partner = my ^ (1 << s)
            rdma = pltpu.make_async_remote_copy(
                src_ref=acc_ref,
                dst_ref=recv_ref.at[s],
                send_sem=send_sems.at[s],
                recv_sem=recv_sems.at[s],
                device_id=(partner,),
                device_id_type=pl.DeviceIdType.MESH,
            )
            rdma.start()
            rdma.wait()
            acc_ref[...] = acc_ref[...] + recv_ref[s]

        out_ref[...] = acc_ref[...]

        @functools.partial(
            pl.run_scoped, second_barrier=pltpu.SemaphoreType.REGULAR)
        def _(second_barrier):
            for s in range(N_STEPS):
                pl.semaphore_signal(
                    second_barrier, inc=1,
                    device_id=(my ^ (1 << s),),
                    device_id_type=pl.DeviceIdType.MESH,
                )
            pl.semaphore_wait(second_barrier, N_STEPS)

    return pl.pallas_call(
        body,
        out_shape=jax.ShapeDtypeStruct((B, SQ, D_MODEL), jnp.float32),
        in_specs=[pl.BlockSpec(memory_space=pltpu.VMEM)] * 5,
        out_specs=pl.BlockSpec(memory_space=pltpu.VMEM),
        scratch_shapes=[
            pltpu.VMEM((B, SQ, D_MODEL), jnp.float32),
            pltpu.VMEM((N_STEPS, B, SQ, D_MODEL), jnp.float32),
            pltpu.SemaphoreType.DMA((N_STEPS,)),
            pltpu.SemaphoreType.DMA((N_STEPS,)),
        ],
        compiler_params=pltpu.CompilerParams(collective_id=0),
    )(x, Wq, K_ext, V_ext, Wo)


# device time: 46604 ns/iter; 1.2006x vs baseline; 1.2006x over previous
import functools

import jax
import jax.numpy as jnp
from jax import lax
from jax.experimental import pallas as pl
from jax.experimental.pallas import tpu as pltpu

N_DEV = 16
B, SQ, SKV = 2, 128, 128
HQ_LOC, DH = 4, 64
D_MODEL = 512
HD_LOC = HQ_LOC * DH
HALF = D_MODEL // 2
MASKS = (1, 3, 4, 8)
N_STEPS = len(MASKS)


def kernel(x, Wq, K_ext, V_ext, Wo):
    def body(x_ref, wq_ref, k_ref, v_ref, wo_ref, out_ref,
             acc_a, acc_b, recv_a, recv_b,
             ssem_a, rsem_a, ssem_b, rsem_b):
        my = lax.axis_index("i")

        barrier_sem = pltpu.get_barrier_semaphore()
        for m in MASKS:
            pl.semaphore_signal(
                barrier_sem, inc=1,
                device_id=(my ^ m,),
                device_id_type=pl.DeviceIdType.MESH,
            )
        pl.semaphore_wait(barrier_sem, N_STEPS)

        col0 = my * HD_LOC
        wq_cols = wq_ref[:, pl.ds(col0, HD_LOC)]
        wo_rows = wo_ref[pl.ds(col0, HD_LOC), :]

        row_blk = lax.broadcasted_iota(jnp.int32, (SQ, SKV), 0) // 64
        col_blk = lax.broadcasted_iota(jnp.int32, (SQ, SKV), 1) // 64
        mask = col_blk <= row_blk

        for b in range(B):
            q_b = jnp.dot(x_ref[b], wq_cols,
                          preferred_element_type=jnp.float32)
            ctx_heads = []
            for h in range(HQ_LOC):
                q = q_b[:, h * DH:(h + 1) * DH]
                k = k_ref[b, :, h, :]
                v = v_ref[b, :, h, :]
                scores = lax.dot_general(
                    q, k, (((1,), (1,)), ((), ())),
                    preferred_element_type=jnp.float32) * 0.125
                scores = jnp.where(mask, scores, -1e9)
                mx = jnp.max(scores, axis=1, keepdims=True)
                w = jnp.exp(scores - mx)
                w = w / jnp.sum(w, axis=1, keepdims=True)
                ctx_heads.append(
                    jnp.dot(w, v, preferred_element_type=jnp.float32))
            ctx_b = jnp.concatenate(ctx_heads, axis=1)
            acc_a[b] = jnp.dot(ctx_b, wo_rows[:, :HALF],
                               preferred_element_type=jnp.float32)
            acc_b[b] = jnp.dot(ctx_b, wo_rows[:, HALF:],
                               preferred_element_type=jnp.float32)

        def mk(acc, recv, ssem, rsem, s):
            return pltpu.make_async_remote_copy(
                src_ref=acc,
                dst_ref=recv.at[s],
                send_sem=ssem.at[s],
                recv_sem=rsem.at[s],
                device_id=(my ^ MASKS[s],),
                device_id_type=pl.DeviceIdType.MESH,
            )

        rdma_a = mk(acc_a, recv_a, ssem_a, rsem_a, 0)
        rdma_a.start()
        rdma_b = mk(acc_b, recv_b, ssem_b, rsem_b, 0)
        rdma_b.start()
        for s in range(N_STEPS):
            rdma_a.wait()
            acc_a[...] = acc_a[...] + recv_a[s]
            if s + 1 < N_STEPS:
                rdma_a = mk(acc_a, recv_a, ssem_a, rsem_a, s + 1)
                rdma_a.start()
            rdma_b.wait()
            acc_b[...] = acc_b[...] + recv_b[s]
            if s + 1 < N_STEPS:
                rdma_b = mk(acc_b, recv_b, ssem_b, rsem_b, s + 1)
                rdma_b.start()

        out_ref[:, :, :HALF] = acc_a[...]
        out_ref[:, :, HALF:] = acc_b[...]

        @functools.partial(
            pl.run_scoped, second_barrier=pltpu.SemaphoreType.REGULAR)
        def _(second_barrier):
            for m in MASKS:
                pl.semaphore_signal(
                    second_barrier, inc=1,
                    device_id=(my ^ m,),
                    device_id_type=pl.DeviceIdType.MESH,
                )
            pl.semaphore_wait(second_barrier, N_STEPS)

    return pl.pallas_call(
        body,
        out_shape=jax.ShapeDtypeStruct((B, SQ, D_MODEL), jnp.float32),
        in_specs=[pl.BlockSpec(memory_space=pltpu.VMEM)] * 5,
        out_specs=pl.BlockSpec(memory_space=pltpu.VMEM),
        scratch_shapes=[
            pltpu.VMEM((B, SQ, HALF), jnp.float32),
            pltpu.VMEM((B, SQ, HALF), jnp.float32),
            pltpu.VMEM((N_STEPS, B, SQ, HALF), jnp.float32),
            pltpu.VMEM((N_STEPS, B, SQ, HALF), jnp.float32),
            pltpu.SemaphoreType.DMA((N_STEPS,)),
            pltpu.SemaphoreType.DMA((N_STEPS,)),
            pltpu.SemaphoreType.DMA((N_STEPS,)),
            pltpu.SemaphoreType.DMA((N_STEPS,)),
        ],
        compiler_params=pltpu.CompilerParams(collective_id=0),
    )(x, Wq, K_ext, V_ext, Wo)


# device time: 35112 ns/iter; 1.5936x vs baseline; 1.3273x over previous
import jax
import jax.numpy as jnp
from jax import lax
from jax.experimental import pallas as pl
from jax.experimental.pallas import tpu as pltpu

N_DEV = 16
B, SQ, SKV = 2, 128, 128
HQ_LOC, DH = 4, 64
D_MODEL = 512
HD_LOC = HQ_LOC * DH
ROWS = B * SQ
CHUNK = ROWS // N_DEV


def kernel(x, Wq, K_ext, V_ext, Wo):
    def body(x_ref, wq_ref, k_ref, v_ref, wo_ref, out_ref,
             acc_ref, recv_ref, ssem_rs, rsem_rs, ssem_ag, rsem_ag):
        my = lax.axis_index("i")

        def out_chunk(c):
            return out_ref.at[c // 8, pl.ds((c % 8) * CHUNK, CHUNK), :]

        barrier_sem = pltpu.get_barrier_semaphore()
        for c in range(N_DEV):
            @pl.when(c != my)
            def _():
                pl.semaphore_signal(
                    barrier_sem, inc=1,
                    device_id=(c,), device_id_type=pl.DeviceIdType.MESH,
                )
        pl.semaphore_wait(barrier_sem, N_DEV - 1)

        col0 = my * HD_LOC
        wq_cols = wq_ref[:, pl.ds(col0, HD_LOC)]
        wo_rows = wo_ref[pl.ds(col0, HD_LOC), :]

        row_blk = lax.broadcasted_iota(jnp.int32, (SQ, SKV), 0) // 64
        col_blk = lax.broadcasted_iota(jnp.int32, (SQ, SKV), 1) // 64
        mask = col_blk <= row_blk

        rs_sends = []
        for b in range(B):
            q_b = jnp.dot(x_ref[b], wq_cols,
                          preferred_element_type=jnp.float32)
            ctx_heads = []
            for h in range(HQ_LOC):
                q = q_b[:, h * DH:(h + 1) * DH]
                k = k_ref[b, :, h, :]
                v = v_ref[b, :, h, :]
                scores = lax.dot_general(
                    q, k, (((1,), (1,)), ((), ())),
                    preferred_element_type=jnp.float32) * 0.125
                scores = jnp.where(mask, scores, -1e9)
                mx = jnp.max(scores, axis=1, keepdims=True)
                w = jnp.exp(scores - mx)
                w = w / jnp.sum(w, axis=1, keepdims=True)
                ctx_heads.append(
                    jnp.dot(w, v, preferred_element_type=jnp.float32))
            ctx_b = jnp.concatenate(ctx_heads, axis=1)
            acc_ref[pl.ds(b * SQ, SQ), :] = jnp.dot(
                ctx_b, wo_rows, preferred_element_type=jnp.float32)

            for c in range(b * 8, (b + 1) * 8):
                desc = pltpu.make_async_remote_copy(
                    src_ref=acc_ref.at[pl.ds(c * CHUNK, CHUNK), :],
                    dst_ref=recv_ref.at[my],
                    send_sem=ssem_rs.at[c],
                    recv_sem=rsem_rs.at[my],
                    device_id=(c,),
                    device_id_type=pl.DeviceIdType.MESH,
                )
                rs_sends.append((c, desc))

                @pl.when(c != my)
                def _(desc=desc):
                    desc.start()

        recv_ref[my] = acc_ref[pl.ds(my * CHUNK, CHUNK), :]

        for c in range(N_DEV):
            @pl.when(c != my)
            def _(c=c):
                pltpu.make_async_remote_copy(
                    src_ref=acc_ref.at[pl.ds(c * CHUNK, CHUNK), :],
                    dst_ref=recv_ref.at[c],
                    send_sem=ssem_rs.at[c],
                    recv_sem=rsem_rs.at[c],
                    device_id=(c,),
                    device_id_type=pl.DeviceIdType.MESH,
                ).wait_recv()

        red = jnp.sum(recv_ref[...], axis=0)
        out_chunk(my)[...] = red

        ag_sends = []
        for c in range(N_DEV):
            desc = pltpu.make_async_remote_copy(
                src_ref=out_chunk(my),
                dst_ref=out_chunk(my),
                send_sem=ssem_ag.at[c],
                recv_sem=rsem_ag.at[my],
                device_id=(c,),
                device_id_type=pl.DeviceIdType.MESH,
            )
            ag_sends.append((c, desc))

            @pl.when(c != my)
            def _(desc=desc):
                desc.start()

        for c in range(N_DEV):
            @pl.when(c != my)
            def _(c=c):
                pltpu.make_async_remote_copy(
                    src_ref=out_chunk(my),
                    dst_ref=out_chunk(c),
                    send_sem=ssem_ag.at[c],
                    recv_sem=rsem_ag.at[c],
                    device_id=(c,),
                    device_id_type=pl.DeviceIdType.MESH,
                ).wait_recv()

        for c, desc in rs_sends + ag_sends:
            @pl.when(c != my)
            def _(desc=desc):
                desc.wait_send()

    return pl.pallas_call(
        body,
        out_shape=jax.ShapeDtypeStruct((B, SQ, D_MODEL), jnp.float32),
        in_specs=[pl.BlockSpec(memory_space=pltpu.VMEM)] * 5,
        out_specs=pl.BlockSpec(memory_space=pltpu.VMEM),
        scratch_shapes=[
            pltpu.VMEM((ROWS, D_MODEL), jnp.float32),
            pltpu.VMEM((N_DEV, CHUNK, D_MODEL), jnp.float32),
            pltpu.SemaphoreType.DMA((N_DEV,)),
            pltpu.SemaphoreType.DMA((N_DEV,)),
            pltpu.SemaphoreType.DMA((N_DEV,)),
            pltpu.SemaphoreType.DMA((N_DEV,)),
        ],
        compiler_params=pltpu.CompilerParams(collective_id=0),
    )(x, Wq, K_ext, V_ext, Wo)
